# baseline (speedup 1.0000x reference)
import functools

import jax
import jax.numpy as jnp
from jax import lax
from jax.experimental import pallas as pl
from jax.experimental.pallas import tpu as pltpu
from jax.experimental.pallas import tpu_sc as plsc

_N = 10000
_E = 320000
_D = 128

_NC = 2
_NS = 16
_NW = _NC * _NS
_GROUP = 128
_GPT = 80
_NGP = _NW * _GPT
_EPAD = _NGP * _GROUP
_RPT = 632
_NPAD = _NS * _RPT


def _sc_spmm(x, row, col, w, zeros):
    mesh = plsc.VectorSubcoreMesh(core_axis_name="c", subcore_axis_name="s")

    @functools.partial(
        pl.kernel,
        mesh=mesh,
        out_type=jax.ShapeDtypeStruct((_NC, _NPAD, _D), jnp.float32),
        scratch_types=[
            pltpu.VMEM((_GROUP,), jnp.int32),
            pltpu.VMEM((_GROUP,), jnp.int32),
            pltpu.VMEM((_GROUP,), jnp.float32),
            pltpu.VMEM((_GROUP,), jnp.int32),
            pltpu.VMEM((_GROUP,), jnp.int32),
            pltpu.VMEM((_GROUP,), jnp.float32),
            pltpu.VMEM((_GROUP, _D), jnp.float32),
            pltpu.VMEM((_GROUP, _D), jnp.float32),
            pltpu.VMEM_SHARED((_NPAD, _D), jnp.float32),
            pltpu.SemaphoreType.DMA,
            pltpu.SemaphoreType.DMA,
            pltpu.SemaphoreType.DMA,
            pltpu.SemaphoreType.DMA,
        ],
    )
    def k(x_hbm, row_hbm, col_hbm, w_hbm, z_hbm, out_hbm,
          col0, row0, w0, col1, row1, w1, rows0, rows1, acc_sh,
          si0, si1, sg0, sg1):
        cid = lax.axis_index("c")
        sid = lax.axis_index("s")
        wid = sid * _NC + cid

        def idx_fetch(t, cbuf, rbuf, wbuf, sem):
            base = (wid + t * _NW) * _GROUP
            pltpu.async_copy(col_hbm.at[pl.ds(base, _GROUP)], cbuf, sem)
            pltpu.async_copy(row_hbm.at[pl.ds(base, _GROUP)], rbuf, sem)
            pltpu.async_copy(w_hbm.at[pl.ds(base, _GROUP)], wbuf, sem)

        def idx_wait(cbuf, rbuf, wbuf, sem):
            pltpu.make_async_copy(col_hbm.at[pl.ds(0, _GROUP)], cbuf,
                                  sem).wait()
            pltpu.make_async_copy(row_hbm.at[pl.ds(0, _GROUP)], rbuf,
                                  sem).wait()
            pltpu.make_async_copy(w_hbm.at[pl.ds(0, _GROUP)], wbuf,
                                  sem).wait()

        def gather(cbuf, rows, sem):
            pltpu.async_copy(x_hbm.at[cbuf], rows, sem)

        def gwait(sem, buf):
            pltpu.make_async_copy(x_hbm.at[col0], buf, sem).wait()

        def scale(rows, wbuf):
            def escale(s, c2):
                wv16 = wbuf[pl.ds(s * 16, 16)]
                for j in range(16):
                    e = s * 16 + j
                    wv = jnp.full((16,), wv16[j], dtype=jnp.float32)
                    for dd in range(_D // 16):
                        sl = pl.ds(dd * 16, 16)
                        rows[e, sl] = rows[e, sl] * wv
                return c2

            lax.fori_loop(0, _GROUP // 16, escale, 0)

        def scatter(rows, rbuf):
            pltpu.sync_copy(rows, acc_sh.at[rbuf], add=True)

        idx_fetch(0, col0, row0, w0, si0)
        idx_wait(col0, row0, w0, si0)
        idx_fetch(1, col1, row1, w1, si1)
        gather(col0, rows0, sg0)
        pltpu.sync_copy(z_hbm, acc_sh.at[pl.ds(sid * _RPT, _RPT)])
        plsc.subcore_barrier()

        def body2(t, carry):
            idx_wait(col1, row1, w1, si1)
            gather(col1, rows1, sg1)
            gwait(sg0, rows0)
            scale(rows0, w0)
            scatter(rows0, row0)
            idx_fetch(2 * t + 2, col0, row0, w0, si0)
            idx_wait(col0, row0, w0, si0)
            gather(col0, rows0, sg0)
            gwait(sg1, rows1)
            scale(rows1, w1)
            scatter(rows1, row1)
            idx_fetch(2 * t + 3, col1, row1, w1, si1)
            return carry

        lax.fori_loop(0, _GPT // 2 - 1, body2, 0)

        idx_wait(col1, row1, w1, si1)
        gather(col1, rows1, sg1)
        gwait(sg0, rows0)
        scale(rows0, w0)
        scatter(rows0, row0)
        gwait(sg1, rows1)
        scale(rows1, w1)
        scatter(rows1, row1)

        plsc.subcore_barrier()
        pltpu.sync_copy(acc_sh.at[pl.ds(sid * _RPT, _RPT)],
                        out_hbm.at[cid, pl.ds(sid * _RPT, _RPT)])

    return k(x, row, col, w, zeros)


def _add_body(a_ref, o_ref):
    o_ref[...] = a_ref[0] + a_ref[1]


def _combine(partials):
    blk = 1000
    return pl.pallas_call(
        _add_body,
        grid=(_N // blk,),
        in_specs=[pl.BlockSpec((_NC, blk, _D), lambda i: (0, i, 0))],
        out_specs=pl.BlockSpec((blk, _D), lambda i: (i, 0)),
        out_shape=jax.ShapeDtypeStruct((_N, _D), jnp.float32),
    )(partials)


def kernel(x, edge_index, edge_weight):
    pad = _EPAD - _E
    row = jnp.concatenate([edge_index[0], jnp.zeros((pad,), jnp.int32)])
    col = jnp.concatenate([edge_index[1], jnp.zeros((pad,), jnp.int32)])
    w = jnp.concatenate([edge_weight, jnp.zeros((pad,), jnp.float32)])
    zeros = jnp.zeros((_RPT, _D), jnp.float32)
    partials = _sc_spmm(x, row, col, w, zeros)
    return _combine(partials[:, :_N])


# speedup vs baseline: 1.4996x; 1.4996x over previous
import functools

import jax
import jax.numpy as jnp
from jax import lax
from jax.experimental import pallas as pl
from jax.experimental.pallas import tpu as pltpu
from jax.experimental.pallas import tpu_sc as plsc

_N = 10000
_E = 320000
_D = 128

_NC = 2
_NS = 16
_NW = _NC * _NS
_GROUP = 128
_NGROUPS = _E // _GROUP
_RPT = 632
_NPAD = _NS * _RPT


def _sc_spmm(x, packed, w, zeros):
    mesh = plsc.VectorSubcoreMesh(core_axis_name="c", subcore_axis_name="s")

    @functools.partial(
        pl.kernel,
        mesh=mesh,
        out_type=jax.ShapeDtypeStruct((_NC, _NPAD, _D), jnp.float32),
        scratch_types=[
            pltpu.VMEM((2, _GROUP), jnp.int32),
            pltpu.VMEM((_GROUP,), jnp.float32),
            pltpu.VMEM((_GROUP, _D), jnp.float32),
            pltpu.VMEM_SHARED((_NPAD, _D), jnp.float32),
            pltpu.SemaphoreType.DMA,
        ],
    )
    def k(x_hbm, pk_hbm, w_hbm, z_hbm, out_hbm,
          pk_v, w_v, rows_v, acc_sh, sem):
        cid = lax.axis_index("c")
        sid = lax.axis_index("s")
        wid = sid * _NC + cid

        pltpu.sync_copy(z_hbm, acc_sh.at[pl.ds(sid * _RPT, _RPT)])
        plsc.subcore_barrier()

        n_mine = (_NGROUPS - wid + _NW - 1) // _NW

        def group_body(t, carry):
            g = wid + t * _NW
            pltpu.sync_copy(pk_hbm.at[g], pk_v)
            pltpu.sync_copy(w_hbm.at[pl.ds(g * _GROUP, _GROUP)], w_v)
            pltpu.async_copy(x_hbm.at[pk_v.at[0]], rows_v, sem).wait()

            def escale(s, c2):
                wv16 = w_v[pl.ds(s * 16, 16)]
                for j in range(16):
                    e = s * 16 + j
                    wv = jnp.full((16,), wv16[j], dtype=jnp.float32)
                    for dd in range(_D // 16):
                        sl = pl.ds(dd * 16, 16)
                        rows_v[e, sl] = rows_v[e, sl] * wv
                return c2

            lax.fori_loop(0, _GROUP // 16, escale, 0)
            pltpu.sync_copy(rows_v, acc_sh.at[pk_v.at[1]], add=True)
            return carry

        lax.fori_loop(0, n_mine, group_body, 0)
        plsc.subcore_barrier()
        pltpu.sync_copy(acc_sh.at[pl.ds(sid * _RPT, _RPT)],
                        out_hbm.at[cid, pl.ds(sid * _RPT, _RPT)])

    return k(x, packed, w, zeros)


def _add_body(a_ref, o_ref):
    o_ref[...] = a_ref[0] + a_ref[1]


def _combine(partials):
    blk = 1000
    return pl.pallas_call(
        _add_body,
        grid=(_N // blk,),
        in_specs=[pl.BlockSpec((_NC, blk, _D), lambda i: (0, i, 0))],
        out_specs=pl.BlockSpec((blk, _D), lambda i: (i, 0)),
        out_shape=jax.ShapeDtypeStruct((_N, _D), jnp.float32),
    )(partials)


def kernel(x, edge_index, edge_weight):
    packed = jnp.stack([
        edge_index[1].reshape(_NGROUPS, _GROUP),
        edge_index[0].reshape(_NGROUPS, _GROUP)],
        axis=1)
    zeros = jnp.zeros((_RPT, _D), jnp.float32)
    partials = _sc_spmm(x, packed, edge_weight, zeros)
    return _combine(partials)
